# Initial kernel scaffold; baseline (speedup 1.0000x reference)
import jax
import jax.numpy as jnp
from jax.experimental import pallas as pl


def kernel(x, edge_index, edge_weights, batch, W1, b1, W2, b2):
    raise NotImplementedError("write your pallas kernel here")


# speedup vs baseline: 10.1798x; 10.1798x over previous
import functools

import jax
import jax.numpy as jnp
from jax import lax
from jax.experimental import pallas as pl
from jax.experimental.pallas import tpu as pltpu
from jax.experimental.pallas import tpu_sc as plsc

N = 10000
E = 160000
D_IN = 256
D_H = 512
D_OUT = 256
N_GRAPHS = 32

NC = 2
NS = 16
HALF = 128

def _mesh():
    return plsc.VectorSubcoreMesh(
        core_axis_name="c", subcore_axis_name="s", num_cores=NC, num_subcores=NS
    )


_RPS = 624
_RPS_LAST = N - 15 * _RPS


_DEG_CH = 1000
_DEG_PER_W = E // (NC * NS)


def _deg_sc(dst, ones_rows, zeros16):
    return pl.kernel(
        _deg_sc_body,
        mesh=_mesh(),
        out_type=jax.ShapeDtypeStruct((NC, N), jnp.float32),
        scratch_types=[
            pltpu.VMEM((_DEG_CH,), jnp.int32),
            pltpu.VMEM((_DEG_CH,), jnp.float32),
            pltpu.VMEM_SHARED((N,), jnp.float32),
        ],
    )(dst, ones_rows, zeros16)


def _deg_sc_body(dst_hbm, ones_hbm, z16_hbm, out_hbm, idx_v, ones_v, acc):
    c = lax.axis_index("c")
    s = lax.axis_index("s")
    wid = c * NS + s

    @pl.when(s == 0)
    def _():
        pltpu.sync_copy(z16_hbm, acc)

    pltpu.sync_copy(ones_hbm, ones_v)
    plsc.subcore_barrier()
    base = wid * _DEG_PER_W

    @pl.loop(0, _DEG_PER_W // _DEG_CH)
    def _(i):
        pltpu.sync_copy(dst_hbm.at[pl.ds(base + i * _DEG_CH, _DEG_CH)], idx_v)
        pltpu.sync_copy(ones_v, acc.at[idx_v], add=True)

    plsc.subcore_barrier()

    @pl.when(s == 0)
    def _():
        pltpu.sync_copy(acc, out_hbm.at[c])


_AGG_CH = 400
_AGG_PER_W = E // NS

_NH = N // 2
_ACC_R = _NH + 8


def _agg_sc(xs_flat, src, dst, zeros_big):
    return pl.kernel(
        _agg_sc_body,
        mesh=_mesh(),
        out_type=jax.ShapeDtypeStruct((NC, N, HALF), jnp.float32),
        scratch_types=[
            pltpu.VMEM((_AGG_CH,), jnp.int32),
            pltpu.VMEM((_AGG_CH,), jnp.int32),
            pltpu.VMEM((_AGG_CH, HALF), jnp.float32),
            pltpu.VMEM_SHARED((_ACC_R, HALF), jnp.float32),
            pltpu.SemaphoreType.DMA,
        ],
    )(xs_flat, src, dst, zeros_big)


def _slab_16(s, n, fn):
    per = (n // NS) & ~7
    last = n - 15 * per

    @pl.when(s < NS - 1)
    def _():
        fn(pl.multiple_of(s * per, 8), per)

    @pl.when(s == NS - 1)
    def _():
        fn((NS - 1) * per, last)


def _agg_sc_body(xs_hbm, src_hbm, dst_hbm, z_hbm, out_hbm,
                 src_v, dst_v, rows_v, acc, sem):
    c = lax.axis_index("c")
    s = lax.axis_index("s")
    base = s * _AGG_PER_W
    off = c * N

    for h in (0, 1):
        def init(b, sz):
            pltpu.sync_copy(z_hbm.at[pl.ds(b, sz)], acc.at[pl.ds(b, sz)])

        _slab_16(s, _ACC_R, init)
        plsc.subcore_barrier()

        @pl.loop(0, _AGG_PER_W // _AGG_CH)
        def _(i):
            e0 = base + i * _AGG_CH
            pltpu.sync_copy(src_hbm.at[pl.ds(e0, _AGG_CH)], src_v)
            pltpu.sync_copy(dst_hbm.at[pl.ds(e0, _AGG_CH)], dst_v)
            trash = _NH + (lax.iota(jnp.int32, 16) & 7)

            @pl.loop(0, _AGG_CH, step=16)
            def _(j):
                src_v[pl.ds(j, 16)] = src_v[pl.ds(j, 16)] + off
                dv = dst_v[pl.ds(j, 16)] - (h * _NH)
                ok = (dv >= 0) & (dv < _NH)
                dst_v[pl.ds(j, 16)] = jnp.where(ok, dv, trash)

            pltpu.async_copy(xs_hbm.at[src_v], rows_v, sem).wait()
            pltpu.sync_copy(rows_v, acc.at[dst_v], add=True)

        plsc.subcore_barrier()

        def writeout(b, sz):
            pltpu.sync_copy(acc.at[pl.ds(b, sz)],
                            out_hbm.at[c].at[pl.ds(h * _NH + b, sz)])

        _slab_16(s, _NH, writeout)
        plsc.subcore_barrier()


_BN = 1000
_GRID = N // _BN


def _pass1_body(degp_ref, x_ref, xs_ref, dis_ref):
    dd = degp_ref[...]
    deg = dd[0, :, 0] + dd[1, :, 0] + 1.0
    dis = lax.rsqrt(deg)
    disc = dis[:, None]
    dis_ref[...] = disc
    xs = x_ref[...] * disc
    xs_ref[...] = jnp.transpose(xs.reshape(_BN, 2, HALF), (1, 0, 2))


def _pass1(degpart, x):
    return pl.pallas_call(
        _pass1_body,
        grid=(_GRID,),
        in_specs=[
            pl.BlockSpec((NC, _BN, 1), lambda i: (0, i, 0)),
            pl.BlockSpec((_BN, D_IN), lambda i: (i, 0)),
        ],
        out_specs=[
            pl.BlockSpec((NC, _BN, HALF), lambda i: (0, i, 0)),
            pl.BlockSpec((_BN, 1), lambda i: (i, 0)),
        ],
        out_shape=[
            jax.ShapeDtypeStruct((NC, N, HALF), jnp.float32),
            jax.ShapeDtypeStruct((N, 1), jnp.float32),
        ],
    )(degpart, x)


def _pass2_body(agg_ref, xs_ref, dis_ref, w1_ref, b1_ref, w2_ref, out_ref):
    d = dis_ref[...]
    d3 = d[None, :, :]
    z3 = (agg_ref[...] + xs_ref[...]) * d3
    z = jnp.concatenate([z3[0], z3[1]], axis=1)
    h1 = jnp.maximum(
        jnp.dot(z, w1_ref[...], preferred_element_type=jnp.float32)
        + b1_ref[...], 0.0)
    h2 = jnp.dot(h1, w2_ref[...], preferred_element_type=jnp.float32)
    xs2 = h2 * d
    out_ref[...] = jnp.transpose(xs2.reshape(_BN, 2, HALF), (1, 0, 2))


def _pass2(agg1, xs1, dis, W1, b1, W2):
    return pl.pallas_call(
        _pass2_body,
        grid=(_GRID,),
        in_specs=[
            pl.BlockSpec((NC, _BN, HALF), lambda i: (0, i, 0)),
            pl.BlockSpec((NC, _BN, HALF), lambda i: (0, i, 0)),
            pl.BlockSpec((_BN, 1), lambda i: (i, 0)),
            pl.BlockSpec((D_IN, D_H), lambda i: (0, 0)),
            pl.BlockSpec((1, D_H), lambda i: (0, 0)),
            pl.BlockSpec((D_H, D_OUT), lambda i: (0, 0)),
        ],
        out_specs=pl.BlockSpec((NC, _BN, HALF), lambda i: (0, i, 0)),
        out_shape=jax.ShapeDtypeStruct((NC, N, HALF), jnp.float32),
    )(agg1, xs1, dis, W1, b1.reshape(1, D_H), W2)


def _pass3_body(agg_ref, xs_ref, dis_ref, b2_ref, batch_ref,
                h_ref, pooled_ref):
    i = pl.program_id(0)
    d = dis_ref[...]
    d3 = d[None, :, :]
    z3 = (agg_ref[...] + xs_ref[...]) * d3
    hblk = jnp.concatenate([z3[0], z3[1]], axis=1) + b2_ref[...]
    h_ref[...] = hblk
    bt = batch_ref[...].reshape(1, _BN)
    mask = (lax.broadcasted_iota(jnp.int32, (N_GRAPHS, _BN), 0) == bt)
    contrib = jnp.dot(mask.astype(jnp.float32), hblk,
                      preferred_element_type=jnp.float32)

    @pl.when(i == 0)
    def _():
        pooled_ref[...] = contrib

    @pl.when(i > 0)
    def _():
        pooled_ref[...] += contrib


def _pass3(agg2, xs2, dis, b2, batch):
    return pl.pallas_call(
        _pass3_body,
        grid=(_GRID,),
        in_specs=[
            pl.BlockSpec((NC, _BN, HALF), lambda i: (0, i, 0)),
            pl.BlockSpec((NC, _BN, HALF), lambda i: (0, i, 0)),
            pl.BlockSpec((_BN, 1), lambda i: (i, 0)),
            pl.BlockSpec((1, D_OUT), lambda i: (0, 0)),
            pl.BlockSpec((_BN, 1), lambda i: (i, 0)),
        ],
        out_specs=[
            pl.BlockSpec((_BN, D_OUT), lambda i: (i, 0)),
            pl.BlockSpec((N_GRAPHS, D_OUT), lambda i: (0, 0)),
        ],
        out_shape=[
            jax.ShapeDtypeStruct((N, D_OUT), jnp.float32),
            jax.ShapeDtypeStruct((N_GRAPHS, D_OUT), jnp.float32),
        ],
    )(agg2, xs2, dis, b2.reshape(1, D_OUT), batch.reshape(N, 1))


def kernel(x, edge_index, edge_weights, batch, W1, b1, W2, b2):
    del edge_weights
    src = edge_index[0]
    dst = edge_index[1]

    zeros_big = jnp.zeros((_ACC_R, HALF), jnp.float32)
    zeros16 = jnp.zeros((N,), jnp.float32)
    ones_rows = jnp.ones((_DEG_CH,), jnp.float32)

    degpart = _deg_sc(dst, ones_rows, zeros16)
    xs1, dis = _pass1(degpart.reshape(NC, N, 1), x)
    agg1 = _agg_sc(xs1.reshape(NC * N, HALF), src, dst, zeros_big)
    xs2 = _pass2(agg1, xs1, dis, W1, b1, W2)
    agg2 = _agg_sc(xs2.reshape(NC * N, HALF), src, dst, zeros_big)
    h, pooled = _pass3(agg2, xs2, dis, b2, batch)
    return (h, pooled)


# speedup vs baseline: 15.0961x; 1.4829x over previous
import functools

import jax
import jax.numpy as jnp
from jax import lax
from jax.experimental import pallas as pl
from jax.experimental.pallas import tpu as pltpu
from jax.experimental.pallas import tpu_sc as plsc

N = 10000
E = 160000
D_IN = 256
D_H = 512
D_OUT = 256
N_GRAPHS = 32

NC = 2
NS = 16
HALF = 128

def _mesh():
    return plsc.VectorSubcoreMesh(
        core_axis_name="c", subcore_axis_name="s", num_cores=NC, num_subcores=NS
    )


_RPS = 624
_RPS_LAST = N - 15 * _RPS


def _acc_slab(s, hbm_like, acc, to_acc):
    def do(base, size):
        if to_acc:
            pltpu.sync_copy(hbm_like.at[pl.ds(base, size)],
                            acc.at[pl.ds(base, size)])
        else:
            pltpu.sync_copy(acc.at[pl.ds(base, size)],
                            hbm_like.at[pl.ds(base, size)])

    @pl.when(s < NS - 1)
    def _():
        do(pl.multiple_of(s * _RPS, 8), _RPS)

    @pl.when(s == NS - 1)
    def _():
        do((NS - 1) * _RPS, _RPS_LAST)

_DEG_CH = 1000
_DEG_PER_W = E // (NC * NS)


def _deg_sc(dst, ones_rows, zeros16):
    return pl.kernel(
        _deg_sc_body,
        mesh=_mesh(),
        out_type=jax.ShapeDtypeStruct((NC, N), jnp.float32),
        scratch_types=[
            pltpu.VMEM((_DEG_CH,), jnp.int32),
            pltpu.VMEM((_DEG_CH,), jnp.float32),
            pltpu.VMEM_SHARED((N,), jnp.float32),
        ],
    )(dst, ones_rows, zeros16)


def _deg_sc_body(dst_hbm, ones_hbm, z16_hbm, out_hbm, idx_v, ones_v, acc):
    c = lax.axis_index("c")
    s = lax.axis_index("s")
    wid = c * NS + s

    @pl.when(s == 0)
    def _():
        pltpu.sync_copy(z16_hbm, acc)

    pltpu.sync_copy(ones_hbm, ones_v)
    plsc.subcore_barrier()
    base = wid * _DEG_PER_W

    @pl.loop(0, _DEG_PER_W // _DEG_CH)
    def _(i):
        pltpu.sync_copy(dst_hbm.at[pl.ds(base + i * _DEG_CH, _DEG_CH)], idx_v)
        pltpu.sync_copy(ones_v, acc.at[idx_v], add=True)

    plsc.subcore_barrier()

    @pl.when(s == 0)
    def _():
        pltpu.sync_copy(acc, out_hbm.at[c])


_AGG_CH = 200
_AGG_PER_W = E // NS


def _agg_sc(xs_flat, src, dst, zeros_big):
    return pl.kernel(
        _agg_sc_body,
        mesh=_mesh(),
        out_type=jax.ShapeDtypeStruct((NC, N, HALF), jnp.float32),
        scratch_types=[
            pltpu.VMEM((_AGG_CH,), jnp.int32),
            pltpu.VMEM((_AGG_CH,), jnp.int32),
            pltpu.VMEM((_AGG_CH, HALF), jnp.float32),
            pltpu.VMEM_SHARED((N, HALF), jnp.float32),
            pltpu.SemaphoreType.DMA,
        ],
    )(xs_flat, src, dst, zeros_big)


def _slab_16(s, n, fn):
    per = (n // NS) & ~7
    last = n - 15 * per

    @pl.when(s < NS - 1)
    def _():
        fn(pl.multiple_of(s * per, 8), per)

    @pl.when(s == NS - 1)
    def _():
        fn((NS - 1) * per, last)


def _agg_sc_body(xs_hbm, src_hbm, dst_hbm, z_hbm, out_hbm,
                 src_v, dst_v, rows_v, acc, sem):
    c = lax.axis_index("c")
    s = lax.axis_index("s")
    base = s * _AGG_PER_W
    off = c * N

    def init(b, sz):
        pltpu.sync_copy(z_hbm.at[pl.ds(b, sz)], acc.at[pl.ds(b, sz)])

    _slab_16(s, N, init)
    plsc.subcore_barrier()

    @pl.loop(0, _AGG_PER_W // _AGG_CH)
    def _(i):
        e0 = base + i * _AGG_CH
        pltpu.sync_copy(src_hbm.at[pl.ds(e0, _AGG_CH)], src_v)
        pltpu.sync_copy(dst_hbm.at[pl.ds(e0, _AGG_CH)], dst_v)

        @pl.loop(0, _AGG_CH, step=16)
        def _(j):
            src_v[pl.ds(j, 16)] = src_v[pl.ds(j, 16)] + off

        pltpu.async_copy(xs_hbm.at[src_v], rows_v, sem).wait()
        pltpu.sync_copy(rows_v, acc.at[dst_v], add=True)

    plsc.subcore_barrier()

    def writeout(b, sz):
        pltpu.sync_copy(acc.at[pl.ds(b, sz)], out_hbm.at[c].at[pl.ds(b, sz)])

    _slab_16(s, N, writeout)


_BN = 1000
_GRID = N // _BN


def _pass1_body(degp_ref, x_ref, xs_ref, dis_ref):
    dd = degp_ref[...]
    deg = dd[0, :, 0] + dd[1, :, 0] + 1.0
    dis = lax.rsqrt(deg)
    disc = dis[:, None]
    dis_ref[...] = disc
    xs = x_ref[...] * disc
    xs_ref[...] = jnp.transpose(xs.reshape(_BN, 2, HALF), (1, 0, 2))


def _pass1(degpart, x):
    return pl.pallas_call(
        _pass1_body,
        grid=(_GRID,),
        in_specs=[
            pl.BlockSpec((NC, _BN, 1), lambda i: (0, i, 0)),
            pl.BlockSpec((_BN, D_IN), lambda i: (i, 0)),
        ],
        out_specs=[
            pl.BlockSpec((NC, _BN, HALF), lambda i: (0, i, 0)),
            pl.BlockSpec((_BN, 1), lambda i: (i, 0)),
        ],
        out_shape=[
            jax.ShapeDtypeStruct((NC, N, HALF), jnp.float32),
            jax.ShapeDtypeStruct((N, 1), jnp.float32),
        ],
    )(degpart, x)


def _pass2_body(agg_ref, xs_ref, dis_ref, w1_ref, b1_ref, w2_ref, out_ref):
    d = dis_ref[...]
    d3 = d[None, :, :]
    z3 = (agg_ref[...] + xs_ref[...]) * d3
    z = jnp.concatenate([z3[0], z3[1]], axis=1)
    h1 = jnp.maximum(
        jnp.dot(z, w1_ref[...], preferred_element_type=jnp.float32)
        + b1_ref[...], 0.0)
    h2 = jnp.dot(h1, w2_ref[...], preferred_element_type=jnp.float32)
    xs2 = h2 * d
    out_ref[...] = jnp.transpose(xs2.reshape(_BN, 2, HALF), (1, 0, 2))


def _pass2(agg1, xs1, dis, W1, b1, W2):
    return pl.pallas_call(
        _pass2_body,
        grid=(_GRID,),
        in_specs=[
            pl.BlockSpec((NC, _BN, HALF), lambda i: (0, i, 0)),
            pl.BlockSpec((NC, _BN, HALF), lambda i: (0, i, 0)),
            pl.BlockSpec((_BN, 1), lambda i: (i, 0)),
            pl.BlockSpec((D_IN, D_H), lambda i: (0, 0)),
            pl.BlockSpec((1, D_H), lambda i: (0, 0)),
            pl.BlockSpec((D_H, D_OUT), lambda i: (0, 0)),
        ],
        out_specs=pl.BlockSpec((NC, _BN, HALF), lambda i: (0, i, 0)),
        out_shape=jax.ShapeDtypeStruct((NC, N, HALF), jnp.float32),
    )(agg1, xs1, dis, W1, b1.reshape(1, D_H), W2)


def _pass3_body(agg_ref, xs_ref, dis_ref, b2_ref, batch_ref,
                h_ref, pooled_ref):
    i = pl.program_id(0)
    d = dis_ref[...]
    d3 = d[None, :, :]
    z3 = (agg_ref[...] + xs_ref[...]) * d3
    hblk = jnp.concatenate([z3[0], z3[1]], axis=1) + b2_ref[...]
    h_ref[...] = hblk
    bt = batch_ref[...].reshape(1, _BN)
    mask = (lax.broadcasted_iota(jnp.int32, (N_GRAPHS, _BN), 0) == bt)
    contrib = jnp.dot(mask.astype(jnp.float32), hblk,
                      preferred_element_type=jnp.float32)

    @pl.when(i == 0)
    def _():
        pooled_ref[...] = contrib

    @pl.when(i > 0)
    def _():
        pooled_ref[...] += contrib


def _pass3(agg2, xs2, dis, b2, batch):
    return pl.pallas_call(
        _pass3_body,
        grid=(_GRID,),
        in_specs=[
            pl.BlockSpec((NC, _BN, HALF), lambda i: (0, i, 0)),
            pl.BlockSpec((NC, _BN, HALF), lambda i: (0, i, 0)),
            pl.BlockSpec((_BN, 1), lambda i: (i, 0)),
            pl.BlockSpec((1, D_OUT), lambda i: (0, 0)),
            pl.BlockSpec((_BN, 1), lambda i: (i, 0)),
        ],
        out_specs=[
            pl.BlockSpec((_BN, D_OUT), lambda i: (i, 0)),
            pl.BlockSpec((N_GRAPHS, D_OUT), lambda i: (0, 0)),
        ],
        out_shape=[
            jax.ShapeDtypeStruct((N, D_OUT), jnp.float32),
            jax.ShapeDtypeStruct((N_GRAPHS, D_OUT), jnp.float32),
        ],
    )(agg2, xs2, dis, b2.reshape(1, D_OUT), batch.reshape(N, 1))


def kernel(x, edge_index, edge_weights, batch, W1, b1, W2, b2):
    del edge_weights
    src = edge_index[0]
    dst = edge_index[1]

    zeros_big = jnp.zeros((N, HALF), jnp.float32)
    zeros16 = jnp.zeros((N,), jnp.float32)
    ones_rows = jnp.ones((_DEG_CH,), jnp.float32)

    degpart = _deg_sc(dst, ones_rows, zeros16)
    xs1, dis = _pass1(degpart.reshape(NC, N, 1), x)
    agg1 = _agg_sc(xs1.reshape(NC * N, HALF), src, dst, zeros_big)
    xs2 = _pass2(agg1, xs1, dis, W1, b1, W2)
    agg2 = _agg_sc(xs2.reshape(NC * N, HALF), src, dst, zeros_big)
    h, pooled = _pass3(agg2, xs2, dis, b2, batch)
    return (h, pooled)


# speedup vs baseline: 16.1562x; 1.0702x over previous
import functools

import jax
import jax.numpy as jnp
from jax import lax
from jax.experimental import pallas as pl
from jax.experimental.pallas import tpu as pltpu
from jax.experimental.pallas import tpu_sc as plsc

N = 10000
E = 160000
D_IN = 256
D_H = 512
D_OUT = 256
N_GRAPHS = 32

NC = 2
NS = 16
HALF = 128

def _mesh():
    return plsc.VectorSubcoreMesh(
        core_axis_name="c", subcore_axis_name="s", num_cores=NC, num_subcores=NS
    )


_RPS = 624
_RPS_LAST = N - 15 * _RPS


def _acc_slab(s, hbm_like, acc, to_acc):
    def do(base, size):
        if to_acc:
            pltpu.sync_copy(hbm_like.at[pl.ds(base, size)],
                            acc.at[pl.ds(base, size)])
        else:
            pltpu.sync_copy(acc.at[pl.ds(base, size)],
                            hbm_like.at[pl.ds(base, size)])

    @pl.when(s < NS - 1)
    def _():
        do(pl.multiple_of(s * _RPS, 8), _RPS)

    @pl.when(s == NS - 1)
    def _():
        do((NS - 1) * _RPS, _RPS_LAST)

_DEG_CH = 1000
_DEG_PER_W = E // (NC * NS)


def _deg_sc(dst, ones_rows, zeros16):
    return pl.kernel(
        _deg_sc_body,
        mesh=_mesh(),
        out_type=jax.ShapeDtypeStruct((NC, N), jnp.float32),
        scratch_types=[
            pltpu.VMEM((_DEG_CH,), jnp.int32),
            pltpu.VMEM((_DEG_CH,), jnp.float32),
            pltpu.VMEM_SHARED((N,), jnp.float32),
        ],
    )(dst, ones_rows, zeros16)


def _deg_sc_body(dst_hbm, ones_hbm, z16_hbm, out_hbm, idx_v, ones_v, acc):
    c = lax.axis_index("c")
    s = lax.axis_index("s")
    wid = c * NS + s

    @pl.when(s == 0)
    def _():
        pltpu.sync_copy(z16_hbm, acc)

    pltpu.sync_copy(ones_hbm, ones_v)
    plsc.subcore_barrier()
    base = wid * _DEG_PER_W

    @pl.loop(0, _DEG_PER_W // _DEG_CH)
    def _(i):
        pltpu.sync_copy(dst_hbm.at[pl.ds(base + i * _DEG_CH, _DEG_CH)], idx_v)
        pltpu.sync_copy(ones_v, acc.at[idx_v], add=True)

    plsc.subcore_barrier()

    @pl.when(s == 0)
    def _():
        pltpu.sync_copy(acc, out_hbm.at[c])


_AGG_CH = 80
_AGG_PER_W = E // NS
_N_CHUNKS = _AGG_PER_W // _AGG_CH


def _agg_sc(xs_flat, src, dst, zeros_big):
    return pl.kernel(
        _agg_sc_body,
        mesh=_mesh(),
        out_type=jax.ShapeDtypeStruct((NC, N, HALF), jnp.float32),
        scratch_types=[
            pltpu.VMEM((_AGG_CH,), jnp.int32),
            pltpu.VMEM((_AGG_CH,), jnp.int32),
            pltpu.VMEM((_AGG_CH, HALF), jnp.float32),
            pltpu.VMEM((_AGG_CH,), jnp.int32),
            pltpu.VMEM((_AGG_CH,), jnp.int32),
            pltpu.VMEM((_AGG_CH, HALF), jnp.float32),
            pltpu.VMEM_SHARED((N, HALF), jnp.float32),
            pltpu.SemaphoreType.DMA,
            pltpu.SemaphoreType.DMA,
        ],
    )(xs_flat, src, dst, zeros_big)


def _slab_16(s, n, fn):
    per = (n // NS) & ~7
    last = n - 15 * per

    @pl.when(s < NS - 1)
    def _():
        fn(pl.multiple_of(s * per, 8), per)

    @pl.when(s == NS - 1)
    def _():
        fn((NS - 1) * per, last)


def _agg_sc_body(xs_hbm, src_hbm, dst_hbm, z_hbm, out_hbm,
                 src_va, dst_va, rows_va, src_vb, dst_vb, rows_vb, acc,
                 sem_a, sem_b):
    c = lax.axis_index("c")
    s = lax.axis_index("s")
    base = s * _AGG_PER_W
    off = c * N

    def init(b, sz):
        pltpu.sync_copy(z_hbm.at[pl.ds(b, sz)], acc.at[pl.ds(b, sz)])

    _slab_16(s, N, init)
    plsc.subcore_barrier()

    def load_start(i, src_v, dst_v, rows_v, sem):
        e0 = base + i * _AGG_CH
        pltpu.sync_copy(src_hbm.at[pl.ds(e0, _AGG_CH)], src_v)
        pltpu.sync_copy(dst_hbm.at[pl.ds(e0, _AGG_CH)], dst_v)

        @pl.loop(0, _AGG_CH, step=16)
        def _(j):
            src_v[pl.ds(j, 16)] = src_v[pl.ds(j, 16)] + off

        pltpu.async_copy(xs_hbm.at[src_v], rows_v, sem)

    def finish(rows_v, dst_v, sem):
        pltpu.make_async_copy(xs_hbm.at[pl.ds(0, _AGG_CH)],
                              rows_v, sem).wait()
        pltpu.sync_copy(rows_v, acc.at[dst_v], add=True)

    load_start(0, src_va, dst_va, rows_va, sem_a)

    @pl.loop(0, (_N_CHUNKS - 1) // 2)
    def _(t):
        load_start(2 * t + 1, src_vb, dst_vb, rows_vb, sem_b)
        finish(rows_va, dst_va, sem_a)
        load_start(2 * t + 2, src_va, dst_va, rows_va, sem_a)
        finish(rows_vb, dst_vb, sem_b)

    finish(rows_va, dst_va, sem_a)
    plsc.subcore_barrier()

    def writeout(b, sz):
        pltpu.sync_copy(acc.at[pl.ds(b, sz)], out_hbm.at[c].at[pl.ds(b, sz)])

    _slab_16(s, N, writeout)


_BN = 1000
_GRID = N // _BN


def _pass1_body(degp_ref, x_ref, xs_ref, dis_ref):
    dd = degp_ref[...]
    deg = dd[0, :, 0] + dd[1, :, 0] + 1.0
    dis = lax.rsqrt(deg)
    disc = dis[:, None]
    dis_ref[...] = disc
    xs = x_ref[...] * disc
    xs_ref[...] = jnp.transpose(xs.reshape(_BN, 2, HALF), (1, 0, 2))


def _pass1(degpart, x):
    return pl.pallas_call(
        _pass1_body,
        grid=(_GRID,),
        in_specs=[
            pl.BlockSpec((NC, _BN, 1), lambda i: (0, i, 0)),
            pl.BlockSpec((_BN, D_IN), lambda i: (i, 0)),
        ],
        out_specs=[
            pl.BlockSpec((NC, _BN, HALF), lambda i: (0, i, 0)),
            pl.BlockSpec((_BN, 1), lambda i: (i, 0)),
        ],
        out_shape=[
            jax.ShapeDtypeStruct((NC, N, HALF), jnp.float32),
            jax.ShapeDtypeStruct((N, 1), jnp.float32),
        ],
    )(degpart, x)


def _pass2_body(agg_ref, xs_ref, dis_ref, w1_ref, b1_ref, w2_ref, out_ref):
    d = dis_ref[...]
    d3 = d[None, :, :]
    z3 = (agg_ref[...] + xs_ref[...]) * d3
    z = jnp.concatenate([z3[0], z3[1]], axis=1)
    h1 = jnp.maximum(
        jnp.dot(z, w1_ref[...], preferred_element_type=jnp.float32)
        + b1_ref[...], 0.0)
    h2 = jnp.dot(h1, w2_ref[...], preferred_element_type=jnp.float32)
    xs2 = h2 * d
    out_ref[...] = jnp.transpose(xs2.reshape(_BN, 2, HALF), (1, 0, 2))


def _pass2(agg1, xs1, dis, W1, b1, W2):
    return pl.pallas_call(
        _pass2_body,
        grid=(_GRID,),
        in_specs=[
            pl.BlockSpec((NC, _BN, HALF), lambda i: (0, i, 0)),
            pl.BlockSpec((NC, _BN, HALF), lambda i: (0, i, 0)),
            pl.BlockSpec((_BN, 1), lambda i: (i, 0)),
            pl.BlockSpec((D_IN, D_H), lambda i: (0, 0)),
            pl.BlockSpec((1, D_H), lambda i: (0, 0)),
            pl.BlockSpec((D_H, D_OUT), lambda i: (0, 0)),
        ],
        out_specs=pl.BlockSpec((NC, _BN, HALF), lambda i: (0, i, 0)),
        out_shape=jax.ShapeDtypeStruct((NC, N, HALF), jnp.float32),
    )(agg1, xs1, dis, W1, b1.reshape(1, D_H), W2)


def _pass3_body(agg_ref, xs_ref, dis_ref, b2_ref, batch_ref,
                h_ref, pooled_ref):
    i = pl.program_id(0)
    d = dis_ref[...]
    d3 = d[None, :, :]
    z3 = (agg_ref[...] + xs_ref[...]) * d3
    hblk = jnp.concatenate([z3[0], z3[1]], axis=1) + b2_ref[...]
    h_ref[...] = hblk
    bt = batch_ref[...].reshape(1, _BN)
    mask = (lax.broadcasted_iota(jnp.int32, (N_GRAPHS, _BN), 0) == bt)
    contrib = jnp.dot(mask.astype(jnp.float32), hblk,
                      preferred_element_type=jnp.float32)

    @pl.when(i == 0)
    def _():
        pooled_ref[...] = contrib

    @pl.when(i > 0)
    def _():
        pooled_ref[...] += contrib


def _pass3(agg2, xs2, dis, b2, batch):
    return pl.pallas_call(
        _pass3_body,
        grid=(_GRID,),
        in_specs=[
            pl.BlockSpec((NC, _BN, HALF), lambda i: (0, i, 0)),
            pl.BlockSpec((NC, _BN, HALF), lambda i: (0, i, 0)),
            pl.BlockSpec((_BN, 1), lambda i: (i, 0)),
            pl.BlockSpec((1, D_OUT), lambda i: (0, 0)),
            pl.BlockSpec((_BN, 1), lambda i: (i, 0)),
        ],
        out_specs=[
            pl.BlockSpec((_BN, D_OUT), lambda i: (i, 0)),
            pl.BlockSpec((N_GRAPHS, D_OUT), lambda i: (0, 0)),
        ],
        out_shape=[
            jax.ShapeDtypeStruct((N, D_OUT), jnp.float32),
            jax.ShapeDtypeStruct((N_GRAPHS, D_OUT), jnp.float32),
        ],
    )(agg2, xs2, dis, b2.reshape(1, D_OUT), batch.reshape(N, 1))


def kernel(x, edge_index, edge_weights, batch, W1, b1, W2, b2):
    del edge_weights
    src = edge_index[0]
    dst = edge_index[1]

    zeros_big = jnp.zeros((N, HALF), jnp.float32)
    zeros16 = jnp.zeros((N,), jnp.float32)
    ones_rows = jnp.ones((_DEG_CH,), jnp.float32)

    degpart = _deg_sc(dst, ones_rows, zeros16)
    xs1, dis = _pass1(degpart.reshape(NC, N, 1), x)
    agg1 = _agg_sc(xs1.reshape(NC * N, HALF), src, dst, zeros_big)
    xs2 = _pass2(agg1, xs1, dis, W1, b1, W2)
    agg2 = _agg_sc(xs2.reshape(NC * N, HALF), src, dst, zeros_big)
    h, pooled = _pass3(agg2, xs2, dis, b2, batch)
    return (h, pooled)


# speedup vs baseline: 16.1568x; 1.0000x over previous
import jax
import jax.numpy as jnp
from jax import lax
from jax.experimental import pallas as pl
from jax.experimental.pallas import tpu as pltpu
from jax.experimental.pallas import tpu_sc as plsc

N = 10000
E = 160000
D_IN = 256
D_H = 512
D_OUT = 256
N_GRAPHS = 32

NC = 2
NS = 16
HALF = 128

def _mesh():
    return plsc.VectorSubcoreMesh(
        core_axis_name="c", subcore_axis_name="s", num_cores=NC, num_subcores=NS
    )


_DEG_CH = 1000
_DEG_PER_W = E // (NC * NS)


def _deg_sc(dst, ones_rows, zeros16):
    return pl.kernel(
        _deg_sc_body,
        mesh=_mesh(),
        out_type=jax.ShapeDtypeStruct((NC, N), jnp.float32),
        scratch_types=[
            pltpu.VMEM((_DEG_CH,), jnp.int32),
            pltpu.VMEM((_DEG_CH,), jnp.float32),
            pltpu.VMEM_SHARED((N,), jnp.float32),
        ],
    )(dst, ones_rows, zeros16)


def _deg_sc_body(dst_hbm, ones_hbm, z16_hbm, out_hbm, idx_v, ones_v, acc):
    c = lax.axis_index("c")
    s = lax.axis_index("s")
    wid = c * NS + s

    @pl.when(s == 0)
    def _():
        pltpu.sync_copy(z16_hbm, acc)

    pltpu.sync_copy(ones_hbm, ones_v)
    plsc.subcore_barrier()
    base = wid * _DEG_PER_W

    @pl.loop(0, _DEG_PER_W // _DEG_CH)
    def _(i):
        pltpu.sync_copy(dst_hbm.at[pl.ds(base + i * _DEG_CH, _DEG_CH)], idx_v)
        pltpu.sync_copy(ones_v, acc.at[idx_v], add=True)

    plsc.subcore_barrier()

    @pl.when(s == 0)
    def _():
        pltpu.sync_copy(acc, out_hbm.at[c])


_AGG_CH = 80
_AGG_PER_W = E // NS
_N_CHUNKS = _AGG_PER_W // _AGG_CH


def _agg_sc(xs_flat, src, dst, zeros_big):
    return pl.kernel(
        _agg_sc_body,
        mesh=_mesh(),
        out_type=jax.ShapeDtypeStruct((NC, N, HALF), jnp.float32),
        scratch_types=[
            pltpu.VMEM((_AGG_CH,), jnp.int32),
            pltpu.VMEM((_AGG_CH,), jnp.int32),
            pltpu.VMEM((_AGG_CH, HALF), jnp.float32),
            pltpu.VMEM((_AGG_CH,), jnp.int32),
            pltpu.VMEM((_AGG_CH,), jnp.int32),
            pltpu.VMEM((_AGG_CH, HALF), jnp.float32),
            pltpu.VMEM_SHARED((N, HALF), jnp.float32),
            pltpu.SemaphoreType.DMA,
            pltpu.SemaphoreType.DMA,
        ],
    )(xs_flat, src, dst, zeros_big)


def _slab_16(s, n, fn):
    per = (n // NS) & ~7
    last = n - 15 * per

    @pl.when(s < NS - 1)
    def _():
        fn(pl.multiple_of(s * per, 8), per)

    @pl.when(s == NS - 1)
    def _():
        fn((NS - 1) * per, last)


def _agg_sc_body(xs_hbm, src_hbm, dst_hbm, z_hbm, out_hbm,
                 src_va, dst_va, rows_va, src_vb, dst_vb, rows_vb, acc,
                 sem_a, sem_b):
    c = lax.axis_index("c")
    s = lax.axis_index("s")
    base = s * _AGG_PER_W
    off = c * N

    def init(b, sz):
        pltpu.sync_copy(z_hbm.at[pl.ds(b, sz)], acc.at[pl.ds(b, sz)])

    _slab_16(s, N, init)
    plsc.subcore_barrier()

    def load_start(i, src_v, dst_v, rows_v, sem):
        e0 = base + i * _AGG_CH
        pltpu.sync_copy(src_hbm.at[pl.ds(e0, _AGG_CH)], src_v)
        pltpu.sync_copy(dst_hbm.at[pl.ds(e0, _AGG_CH)], dst_v)

        @pl.loop(0, _AGG_CH, step=16)
        def _(j):
            src_v[pl.ds(j, 16)] = src_v[pl.ds(j, 16)] + off

        pltpu.async_copy(xs_hbm.at[src_v], rows_v, sem)

    def finish(rows_v, dst_v, sem):
        pltpu.make_async_copy(xs_hbm.at[pl.ds(0, _AGG_CH)],
                              rows_v, sem).wait()
        pltpu.sync_copy(rows_v, acc.at[dst_v], add=True)

    load_start(0, src_va, dst_va, rows_va, sem_a)

    @pl.loop(0, (_N_CHUNKS - 1) // 2)
    def _(t):
        load_start(2 * t + 1, src_vb, dst_vb, rows_vb, sem_b)
        finish(rows_va, dst_va, sem_a)
        load_start(2 * t + 2, src_va, dst_va, rows_va, sem_a)
        finish(rows_vb, dst_vb, sem_b)

    finish(rows_va, dst_va, sem_a)
    plsc.subcore_barrier()

    def writeout(b, sz):
        pltpu.sync_copy(acc.at[pl.ds(b, sz)], out_hbm.at[c].at[pl.ds(b, sz)])

    _slab_16(s, N, writeout)


_BN = 1000
_GRID = N // _BN


def _pass1_body(degp_ref, x_ref, xs_ref, dis_ref):
    dd = degp_ref[...]
    deg = dd[0, :, 0] + dd[1, :, 0] + 1.0
    dis = lax.rsqrt(deg)
    disc = dis[:, None]
    dis_ref[...] = disc
    xs = x_ref[...] * disc
    xs_ref[...] = jnp.transpose(xs.reshape(_BN, 2, HALF), (1, 0, 2))


def _pass1(degpart, x):
    return pl.pallas_call(
        _pass1_body,
        grid=(_GRID,),
        in_specs=[
            pl.BlockSpec((NC, _BN, 1), lambda i: (0, i, 0)),
            pl.BlockSpec((_BN, D_IN), lambda i: (i, 0)),
        ],
        out_specs=[
            pl.BlockSpec((NC, _BN, HALF), lambda i: (0, i, 0)),
            pl.BlockSpec((_BN, 1), lambda i: (i, 0)),
        ],
        out_shape=[
            jax.ShapeDtypeStruct((NC, N, HALF), jnp.float32),
            jax.ShapeDtypeStruct((N, 1), jnp.float32),
        ],
    )(degpart, x)


def _pass2_body(agg_ref, xs_ref, dis_ref, w1_ref, b1_ref, w2_ref, out_ref):
    d = dis_ref[...]
    d3 = d[None, :, :]
    z3 = (agg_ref[...] + xs_ref[...]) * d3
    z = jnp.concatenate([z3[0], z3[1]], axis=1)
    h1 = jnp.maximum(
        jnp.dot(z, w1_ref[...], preferred_element_type=jnp.float32)
        + b1_ref[...], 0.0)
    h2 = jnp.dot(h1, w2_ref[...], preferred_element_type=jnp.float32)
    xs2 = h2 * d
    out_ref[...] = jnp.transpose(xs2.reshape(_BN, 2, HALF), (1, 0, 2))


def _pass2(agg1, xs1, dis, W1, b1, W2):
    return pl.pallas_call(
        _pass2_body,
        grid=(_GRID,),
        in_specs=[
            pl.BlockSpec((NC, _BN, HALF), lambda i: (0, i, 0)),
            pl.BlockSpec((NC, _BN, HALF), lambda i: (0, i, 0)),
            pl.BlockSpec((_BN, 1), lambda i: (i, 0)),
            pl.BlockSpec((D_IN, D_H), lambda i: (0, 0)),
            pl.BlockSpec((1, D_H), lambda i: (0, 0)),
            pl.BlockSpec((D_H, D_OUT), lambda i: (0, 0)),
        ],
        out_specs=pl.BlockSpec((NC, _BN, HALF), lambda i: (0, i, 0)),
        out_shape=jax.ShapeDtypeStruct((NC, N, HALF), jnp.float32),
    )(agg1, xs1, dis, W1, b1.reshape(1, D_H), W2)


def _pass3_body(agg_ref, xs_ref, dis_ref, b2_ref, batch_ref,
                h_ref, pooled_ref):
    i = pl.program_id(0)
    d = dis_ref[...]
    d3 = d[None, :, :]
    z3 = (agg_ref[...] + xs_ref[...]) * d3
    hblk = jnp.concatenate([z3[0], z3[1]], axis=1) + b2_ref[...]
    h_ref[...] = hblk
    bt = batch_ref[...].reshape(1, _BN)
    mask = (lax.broadcasted_iota(jnp.int32, (N_GRAPHS, _BN), 0) == bt)
    contrib = jnp.dot(mask.astype(jnp.float32), hblk,
                      preferred_element_type=jnp.float32)

    @pl.when(i == 0)
    def _():
        pooled_ref[...] = contrib

    @pl.when(i > 0)
    def _():
        pooled_ref[...] += contrib


def _pass3(agg2, xs2, dis, b2, batch):
    return pl.pallas_call(
        _pass3_body,
        grid=(_GRID,),
        in_specs=[
            pl.BlockSpec((NC, _BN, HALF), lambda i: (0, i, 0)),
            pl.BlockSpec((NC, _BN, HALF), lambda i: (0, i, 0)),
            pl.BlockSpec((_BN, 1), lambda i: (i, 0)),
            pl.BlockSpec((1, D_OUT), lambda i: (0, 0)),
            pl.BlockSpec((_BN, 1), lambda i: (i, 0)),
        ],
        out_specs=[
            pl.BlockSpec((_BN, D_OUT), lambda i: (i, 0)),
            pl.BlockSpec((N_GRAPHS, D_OUT), lambda i: (0, 0)),
        ],
        out_shape=[
            jax.ShapeDtypeStruct((N, D_OUT), jnp.float32),
            jax.ShapeDtypeStruct((N_GRAPHS, D_OUT), jnp.float32),
        ],
    )(agg2, xs2, dis, b2.reshape(1, D_OUT), batch.reshape(N, 1))


def kernel(x, edge_index, edge_weights, batch, W1, b1, W2, b2):
    del edge_weights
    src = edge_index[0]
    dst = edge_index[1]

    zeros_big = jnp.zeros((N, HALF), jnp.float32)
    zeros16 = jnp.zeros((N,), jnp.float32)
    ones_rows = jnp.ones((_DEG_CH,), jnp.float32)

    degpart = _deg_sc(dst, ones_rows, zeros16)
    xs1, dis = _pass1(degpart.reshape(NC, N, 1), x)
    agg1 = _agg_sc(xs1.reshape(NC * N, HALF), src, dst, zeros_big)
    xs2 = _pass2(agg1, xs1, dis, W1, b1, W2)
    agg2 = _agg_sc(xs2.reshape(NC * N, HALF), src, dst, zeros_big)
    h, pooled = _pass3(agg2, xs2, dis, b2, batch)
    return (h, pooled)
